# Initial kernel scaffold; baseline (speedup 1.0000x reference)
import jax
import jax.numpy as jnp
from jax.experimental import pallas as pl


def kernel(x, edge_index, W1, b1, W2, b2, Wd, bd):
    raise NotImplementedError("write your pallas kernel here")


# speedup vs baseline: 16.1399x; 16.1399x over previous
import functools

import jax
import jax.numpy as jnp
from jax import lax
from jax.experimental import pallas as pl
from jax.experimental.pallas import tpu as pltpu
from jax.experimental.pallas import tpu_sc as plsc

N = 10000
E = 320000
NP = 10240
NC = 2
NS = 16
NW = NC * NS
EPW = E // NW
CH = 128
NFULL = EPW // CH
TAIL = EPW - NFULL * CH
ROWS_PER_TILE = NP // NS
DEGW = 16


def _zero_rows(ref, nrows, ncols):
    def row(i, _):
        def col(j, _):
            ref[i, pl.ds(j * 16, 16)] = jnp.zeros((16,), jnp.float32)
            return 0
        return lax.fori_loop(0, ncols // 16, col, 0)
    lax.fori_loop(0, nrows, row, 0)


def _fill_ones(ref, nrows, ncols):
    def row(i, _):
        def col(j, _):
            ref[i, pl.ds(j * 16, 16)] = jnp.ones((16,), jnp.float32)
            return 0
        return lax.fori_loop(0, ncols // 16, col, 0)
    lax.fori_loop(0, nrows, row, 0)


def _sc_mesh():
    return plsc.VectorSubcoreMesh(core_axis_name="c", subcore_axis_name="s")


def _deg_body(dst_hbm, out_hbm, idx_v, idx_t, hist_v):
    c = lax.axis_index("c")
    s = lax.axis_index("s")
    wid = s * NC + c

    def zb(k, _):
        hist_v[k, pl.ds(0, 16)] = jnp.zeros((16,), jnp.float32)
        return 0
    lax.fori_loop(0, N // 16, zb, 0)

    ebase = wid * EPW
    ones16 = jnp.ones((16,), jnp.float32)

    def scat(iv):
        rows = lax.shift_right_logical(iv, 4)
        cols = lax.bitwise_and(iv, 15)
        plsc.addupdate_scatter(hist_v, [rows, cols], ones16)

    def body(j, _):
        pltpu.sync_copy(dst_hbm.at[pl.ds(ebase + j * CH, CH)], idx_v)
        def inner(k, _):
            scat(idx_v[pl.ds(k * 16, 16)])
            return 0
        return lax.fori_loop(0, CH // 16, inner, 0)
    lax.fori_loop(0, NFULL, body, 0)

    pltpu.sync_copy(dst_hbm.at[pl.ds(ebase + NFULL * CH, TAIL)], idx_t)
    scat(idx_t[pl.ds(0, 16)])

    pltpu.sync_copy(hist_v, out_hbm.at[wid])


def _sc_degree(dst):
    return pl.kernel(
        _deg_body,
        out_type=jax.ShapeDtypeStruct((NW, N // 16, 16), jnp.float32),
        mesh=_sc_mesh(),
        scratch_types=[
            pltpu.VMEM((CH,), jnp.int32),
            pltpu.VMEM((TAIL,), jnp.int32),
            pltpu.VMEM((N // 16, 16), jnp.float32),
        ],
        compiler_params=pltpu.CompilerParams(needs_layout_passes=False),
    )(dst)


def _agg_body(g_hbm, src_hbm, dst_hbm, out_hbm,
              idx_s, idx_d, idx_st, idx_dt, rows_v, acc_sh, sem, *, d):
    c = lax.axis_index("c")
    s = lax.axis_index("s")
    wid = s * NC + c

    _zero_rows(rows_v, CH, d)
    for k in range(ROWS_PER_TILE // CH):
        pltpu.sync_copy(rows_v, acc_sh.at[pl.ds(s * ROWS_PER_TILE + k * CH, CH)])
    plsc.subcore_barrier()

    ebase = wid * EPW

    def body(j, _):
        e0 = ebase + j * CH
        pltpu.sync_copy(src_hbm.at[pl.ds(e0, CH)], idx_s)
        pltpu.sync_copy(dst_hbm.at[pl.ds(e0, CH)], idx_d)
        pltpu.async_copy(g_hbm.at[idx_s], rows_v, sem).wait()
        pltpu.sync_copy(rows_v, acc_sh.at[idx_d], add=True)
        return 0
    lax.fori_loop(0, NFULL, body, 0)

    e0 = ebase + NFULL * CH
    pltpu.sync_copy(src_hbm.at[pl.ds(e0, TAIL)], idx_st)
    pltpu.sync_copy(dst_hbm.at[pl.ds(e0, TAIL)], idx_dt)
    pltpu.async_copy(g_hbm.at[idx_st], rows_v.at[pl.ds(0, TAIL)], sem).wait()
    pltpu.sync_copy(rows_v.at[pl.ds(0, TAIL)], acc_sh.at[idx_dt], add=True)

    plsc.subcore_barrier()

    for k in range(ROWS_PER_TILE // CH):
        base = s * ROWS_PER_TILE + k * CH
        pltpu.sync_copy(acc_sh.at[pl.ds(base, CH)], rows_v)
        pltpu.sync_copy(rows_v, out_hbm.at[c, pl.ds(base, CH)])


def _sc_agg(g, src, dst, d):
    return pl.kernel(
        functools.partial(_agg_body, d=d),
        out_type=jax.ShapeDtypeStruct((NC, NP, d), jnp.float32),
        mesh=_sc_mesh(),
        scratch_types=[
            pltpu.VMEM((CH,), jnp.int32),
            pltpu.VMEM((CH,), jnp.int32),
            pltpu.VMEM((TAIL,), jnp.int32),
            pltpu.VMEM((TAIL,), jnp.int32),
            pltpu.VMEM((CH, d), jnp.float32),
            pltpu.VMEM_SHARED((NP, d), jnp.float32),
            pltpu.SemaphoreType.DMA,
        ],
    )(g, src, dst)


R = 1000


def _dinv_body(dp_ref, dinv_ref):
    deg = jnp.sum(dp_ref[...], axis=0) + 1.0
    dinv_ref[...] = lax.rsqrt(deg)[:, None]


def _dinv(dp):
    return pl.pallas_call(
        _dinv_body,
        out_shape=jax.ShapeDtypeStruct((N, 1), jnp.float32),
    )(dp)


def _tc1_body(dinv_ref, x_ref, w1_ref, g1_ref):
    g1_ref[...] = jnp.dot(x_ref[...], w1_ref[...],
                          preferred_element_type=jnp.float32) * dinv_ref[...]


def _tc1(dinv, x, W1):
    grid = N // R
    return pl.pallas_call(
        _tc1_body,
        grid=(grid,),
        in_specs=[
            pl.BlockSpec((R, 1), lambda i: (i, 0)),
            pl.BlockSpec((R, 128), lambda i: (i, 0)),
            pl.BlockSpec((128, 128), lambda i: (0, 0)),
        ],
        out_specs=pl.BlockSpec((R, 128), lambda i: (i, 0)),
        out_shape=jax.ShapeDtypeStruct((N, 128), jnp.float32),
    )(dinv, x, W1)


def _tc2_body(dinv_ref, agg_ref, g1_ref, b1_ref, w2_ref, g2_ref):
    dinv = dinv_ref[...]
    out1 = dinv * (agg_ref[0] + agg_ref[1] + g1_ref[...]) + b1_ref[...]
    out1 = jnp.maximum(out1, 0.0)
    g2 = jnp.dot(out1, w2_ref[...], preferred_element_type=jnp.float32) * dinv
    g2_ref[...] = jnp.concatenate([g2, jnp.zeros_like(g2)], axis=1)


def _tc2(dinv, agg1, g1, b1, W2):
    grid = N // R
    return pl.pallas_call(
        _tc2_body,
        grid=(grid,),
        in_specs=[
            pl.BlockSpec((R, 1), lambda i: (i, 0)),
            pl.BlockSpec((NC, R, 128), lambda i: (0, i, 0)),
            pl.BlockSpec((R, 128), lambda i: (i, 0)),
            pl.BlockSpec((1, 128), lambda i: (0, 0)),
            pl.BlockSpec((128, 64), lambda i: (0, 0)),
        ],
        out_specs=pl.BlockSpec((R, 128), lambda i: (i, 0)),
        out_shape=jax.ShapeDtypeStruct((N, 128), jnp.float32),
    )(dinv, agg1, g1, b1.reshape(1, 128), W2)


def _tc3_body(dinv_ref, agg_ref, g2_ref, b2_ref, wd_ref, bd_ref, pred_ref):
    dinv = dinv_ref[...]
    z = dinv * (agg_ref[0, :, :64] + agg_ref[1, :, :64] + g2_ref[:, :64]) + b2_ref[...]
    m = jnp.max(z, axis=1, keepdims=True)
    lse = jnp.log(jnp.sum(jnp.exp(z - m), axis=1, keepdims=True)) + m
    embeds = z - lse
    logit = jnp.sum(embeds * wd_ref[...], axis=1, keepdims=True) + bd_ref[0, 0]
    pred_ref[...] = jax.nn.sigmoid(logit)


def _tc3(dinv, agg2, g2, b2, Wd, bd):
    grid = N // R
    return pl.pallas_call(
        _tc3_body,
        grid=(grid,),
        in_specs=[
            pl.BlockSpec((R, 1), lambda i: (i, 0)),
            pl.BlockSpec((NC, R, 128), lambda i: (0, i, 0)),
            pl.BlockSpec((R, 128), lambda i: (i, 0)),
            pl.BlockSpec((1, 64), lambda i: (0, 0)),
            pl.BlockSpec((1, 64), lambda i: (0, 0)),
            pl.BlockSpec((1, 1), lambda i: (0, 0)),
        ],
        out_specs=pl.BlockSpec((R, 1), lambda i: (i, 0)),
        out_shape=jax.ShapeDtypeStruct((N, 1), jnp.float32),
    )(dinv, agg2, g2, b2.reshape(1, 64), Wd.reshape(1, 64), bd.reshape(1, 1))


def kernel(x, edge_index, W1, b1, W2, b2, Wd, bd):
    src = edge_index[0]
    dst = edge_index[1]
    dp = _sc_degree(dst).reshape(NW, N)
    dinv = _dinv(dp)
    g1 = _tc1(dinv, x, W1)
    agg1 = _sc_agg(g1, src, dst, 128)
    g2 = _tc2(dinv, agg1, g1, b1, W2)
    agg2 = _sc_agg(g2, src, dst, 128)
    return _tc3(dinv, agg2, g2, b2, Wd, bd)


# speedup vs baseline: 31.4366x; 1.9478x over previous
import functools

import jax
import jax.numpy as jnp
from jax import lax
from jax.experimental import pallas as pl
from jax.experimental.pallas import tpu as pltpu
from jax.experimental.pallas import tpu_sc as plsc

N = 10000
E = 320000
NP = 10240
NC = 2
NS = 16
NW = NC * NS
EPW = E // NW
CH = 128
NROW = E // CH
RPW = NROW // NW
EXTRA = NROW - NW * RPW
ROWS_PER_TILE = NP // NS
DEGW = 16


def _zero_rows(ref, nrows, ncols):
    def row(i, _):
        def col(j, _):
            ref[i, pl.ds(j * 16, 16)] = jnp.zeros((16,), jnp.float32)
            return 0
        return lax.fori_loop(0, ncols // 16, col, 0)
    lax.fori_loop(0, nrows, row, 0)


def _fill_ones(ref, nrows, ncols):
    def row(i, _):
        def col(j, _):
            ref[i, pl.ds(j * 16, 16)] = jnp.ones((16,), jnp.float32)
            return 0
        return lax.fori_loop(0, ncols // 16, col, 0)
    lax.fori_loop(0, nrows, row, 0)


def _sc_mesh():
    return plsc.VectorSubcoreMesh(core_axis_name="c", subcore_axis_name="s")


def _deg_body(dst_hbm, dstx_hbm, out_hbm, didx, didx_x, hist_v):
    c = lax.axis_index("c")
    s = lax.axis_index("s")
    wid = s * NC + c

    def zb(k, _):
        hist_v[k, pl.ds(0, 16)] = jnp.zeros((16,), jnp.float32)
        return 0
    lax.fori_loop(0, N // 16, zb, 0)

    ones16 = jnp.ones((16,), jnp.float32)

    def scat(iv):
        rows = lax.shift_right_logical(iv, 4)
        cols = lax.bitwise_and(iv, 15)
        plsc.addupdate_scatter(hist_v, [rows, cols], ones16)

    pltpu.sync_copy(dst_hbm.at[wid], didx)

    def body(j, _):
        def inner(k, _):
            scat(didx[j, pl.ds(k * 16, 16)])
            return 0
        return lax.fori_loop(0, CH // 16, inner, 0)
    lax.fori_loop(0, RPW, body, 0)

    @pl.when(wid < EXTRA)
    def _():
        pltpu.sync_copy(dstx_hbm.at[wid], didx_x)
        def inner(k, _):
            scat(didx_x[0, pl.ds(k * 16, 16)])
            return 0
        lax.fori_loop(0, CH // 16, inner, 0)

    pltpu.sync_copy(hist_v, out_hbm.at[wid])


def _sc_degree(dst3, dstx):
    return pl.kernel(
        _deg_body,
        out_type=jax.ShapeDtypeStruct((NW, N // 16, 16), jnp.float32),
        mesh=_sc_mesh(),
        scratch_types=[
            pltpu.VMEM((RPW, CH), jnp.int32),
            pltpu.VMEM((1, CH), jnp.int32),
            pltpu.VMEM((N // 16, 16), jnp.float32),
        ],
        compiler_params=pltpu.CompilerParams(needs_layout_passes=False),
    )(dst3, dstx)


def _agg_body(g_hbm, src_hbm, dst_hbm, srcx_hbm, dstx_hbm, out_hbm,
              sidx, didx0, didx1, sidx_x, didx_x, rows0, rows1, acc_sh,
              sem0, sem1, *, d):
    c = lax.axis_index("c")
    s = lax.axis_index("s")
    wid = s * NC + c

    _zero_rows(rows0, CH, d)
    for k in range(ROWS_PER_TILE // CH):
        pltpu.sync_copy(rows0, acc_sh.at[pl.ds(s * ROWS_PER_TILE + k * CH, CH)])
    plsc.subcore_barrier()

    pltpu.sync_copy(src_hbm.at[wid], sidx)

    def start(j, rows, didx, sem):
        pltpu.async_copy(dst_hbm.at[wid, pl.ds(j, 1)], didx, sem)
        pltpu.async_copy(g_hbm.at[sidx.at[j]], rows, sem)

    def drain(j, rows, didx, sem):
        pltpu.make_async_copy(dst_hbm.at[wid, pl.ds(j, 1)], didx, sem).wait()
        pltpu.make_async_copy(g_hbm.at[sidx.at[j]], rows, sem).wait()
        pltpu.sync_copy(rows, acc_sh.at[didx.at[0]], add=True)

    start(0, rows0, didx0, sem0)

    def body(j2, _):
        j = 2 * j2
        start(j + 1, rows1, didx1, sem1)
        drain(j, rows0, didx0, sem0)

        @pl.when(j2 < RPW // 2 - 1)
        def _():
            start(j + 2, rows0, didx0, sem0)
        drain(j + 1, rows1, didx1, sem1)
        return 0
    lax.fori_loop(0, RPW // 2, body, 0)

    @pl.when(wid < EXTRA)
    def _():
        pltpu.sync_copy(srcx_hbm.at[wid], sidx_x)
        pltpu.sync_copy(dstx_hbm.at[wid], didx_x)
        pltpu.async_copy(g_hbm.at[sidx_x.at[0]], rows0, sem0).wait()
        pltpu.sync_copy(rows0, acc_sh.at[didx_x.at[0]], add=True)

    plsc.subcore_barrier()

    for k in range(ROWS_PER_TILE // CH):
        base = s * ROWS_PER_TILE + k * CH
        pltpu.sync_copy(acc_sh.at[pl.ds(base, CH)], rows0)
        pltpu.sync_copy(rows0, out_hbm.at[c, pl.ds(base, CH)])


def _sc_agg(g, src3, dst3, srcx, dstx, d):
    return pl.kernel(
        functools.partial(_agg_body, d=d),
        out_type=jax.ShapeDtypeStruct((NC, NP, d), jnp.float32),
        mesh=_sc_mesh(),
        scratch_types=[
            pltpu.VMEM((RPW, CH), jnp.int32),
            pltpu.VMEM((1, CH), jnp.int32),
            pltpu.VMEM((1, CH), jnp.int32),
            pltpu.VMEM((1, CH), jnp.int32),
            pltpu.VMEM((1, CH), jnp.int32),
            pltpu.VMEM((CH, d), jnp.float32),
            pltpu.VMEM((CH, d), jnp.float32),
            pltpu.VMEM_SHARED((NP, d), jnp.float32),
            pltpu.SemaphoreType.DMA,
            pltpu.SemaphoreType.DMA,
        ],
    )(g, src3, dst3, srcx, dstx)


R = 1000


def _dinv_body(dp_ref, dinv_ref):
    deg = jnp.sum(dp_ref[...], axis=0) + 1.0
    dinv_ref[...] = lax.rsqrt(deg)[:, None]


def _dinv(dp):
    return pl.pallas_call(
        _dinv_body,
        out_shape=jax.ShapeDtypeStruct((N, 1), jnp.float32),
    )(dp)


def _tc1_body(dinv_ref, x_ref, w1_ref, g1_ref):
    g1_ref[...] = jnp.dot(x_ref[...], w1_ref[...],
                          preferred_element_type=jnp.float32) * dinv_ref[...]


def _tc1(dinv, x, W1):
    grid = N // R
    return pl.pallas_call(
        _tc1_body,
        grid=(grid,),
        in_specs=[
            pl.BlockSpec((R, 1), lambda i: (i, 0)),
            pl.BlockSpec((R, 128), lambda i: (i, 0)),
            pl.BlockSpec((128, 128), lambda i: (0, 0)),
        ],
        out_specs=pl.BlockSpec((R, 128), lambda i: (i, 0)),
        out_shape=jax.ShapeDtypeStruct((N, 128), jnp.float32),
    )(dinv, x, W1)


def _tc2_body(dinv_ref, agg_ref, g1_ref, b1_ref, w2_ref, g2_ref):
    dinv = dinv_ref[...]
    out1 = dinv * (agg_ref[0] + agg_ref[1] + g1_ref[...]) + b1_ref[...]
    out1 = jnp.maximum(out1, 0.0)
    g2 = jnp.dot(out1, w2_ref[...], preferred_element_type=jnp.float32) * dinv
    g2_ref[...] = jnp.concatenate([g2, jnp.zeros_like(g2)], axis=1)


def _tc2(dinv, agg1, g1, b1, W2):
    grid = N // R
    return pl.pallas_call(
        _tc2_body,
        grid=(grid,),
        in_specs=[
            pl.BlockSpec((R, 1), lambda i: (i, 0)),
            pl.BlockSpec((NC, R, 128), lambda i: (0, i, 0)),
            pl.BlockSpec((R, 128), lambda i: (i, 0)),
            pl.BlockSpec((1, 128), lambda i: (0, 0)),
            pl.BlockSpec((128, 64), lambda i: (0, 0)),
        ],
        out_specs=pl.BlockSpec((R, 128), lambda i: (i, 0)),
        out_shape=jax.ShapeDtypeStruct((N, 128), jnp.float32),
    )(dinv, agg1, g1, b1.reshape(1, 128), W2)


def _tc3_body(dinv_ref, agg_ref, g2_ref, b2_ref, wd_ref, bd_ref, pred_ref):
    dinv = dinv_ref[...]
    z = dinv * (agg_ref[0, :, :64] + agg_ref[1, :, :64] + g2_ref[:, :64]) + b2_ref[...]
    m = jnp.max(z, axis=1, keepdims=True)
    lse = jnp.log(jnp.sum(jnp.exp(z - m), axis=1, keepdims=True)) + m
    embeds = z - lse
    logit = jnp.sum(embeds * wd_ref[...], axis=1, keepdims=True) + bd_ref[0, 0]
    pred_ref[...] = jax.nn.sigmoid(logit)


def _tc3(dinv, agg2, g2, b2, Wd, bd):
    grid = N // R
    return pl.pallas_call(
        _tc3_body,
        grid=(grid,),
        in_specs=[
            pl.BlockSpec((R, 1), lambda i: (i, 0)),
            pl.BlockSpec((NC, R, 128), lambda i: (0, i, 0)),
            pl.BlockSpec((R, 128), lambda i: (i, 0)),
            pl.BlockSpec((1, 64), lambda i: (0, 0)),
            pl.BlockSpec((1, 64), lambda i: (0, 0)),
            pl.BlockSpec((1, 1), lambda i: (0, 0)),
        ],
        out_specs=pl.BlockSpec((R, 1), lambda i: (i, 0)),
        out_shape=jax.ShapeDtypeStruct((N, 1), jnp.float32),
    )(dinv, agg2, g2, b2.reshape(1, 64), Wd.reshape(1, 64), bd.reshape(1, 1))


def kernel(x, edge_index, W1, b1, W2, b2, Wd, bd):
    nmain = NW * RPW * CH
    src3 = edge_index[0][:nmain].reshape(NW, RPW, CH)
    dst3 = edge_index[1][:nmain].reshape(NW, RPW, CH)
    srcx = edge_index[0][nmain:].reshape(EXTRA, 1, CH)
    dstx = edge_index[1][nmain:].reshape(EXTRA, 1, CH)
    dp = _sc_degree(dst3, dstx).reshape(NW, N)
    dinv = _dinv(dp)
    g1 = _tc1(dinv, x, W1)
    agg1 = _sc_agg(g1, src3, dst3, srcx, dstx, 128)
    g2 = _tc2(dinv, agg1, g1, b1, W2)
    agg2 = _sc_agg(g2, src3, dst3, srcx, dstx, 128)
    return _tc3(dinv, agg2, g2, b2, Wd, bd)


# speedup vs baseline: 33.2990x; 1.0592x over previous
import functools

import jax
import jax.numpy as jnp
from jax import lax
from jax.experimental import pallas as pl
from jax.experimental.pallas import tpu as pltpu
from jax.experimental.pallas import tpu_sc as plsc

N = 10000
E = 320000
NP = 10240
NC = 2
NS = 16
NW = NC * NS
EPW = E // NW
CH = 128
NROW = E // CH
RPW = NROW // NW
EXTRA = NROW - NW * RPW
ROWS_PER_TILE = NP // NS
DEGW = 16


def _zero_rows(ref, nrows, ncols):
    def row(i, _):
        def col(j, _):
            ref[i, pl.ds(j * 16, 16)] = jnp.zeros((16,), jnp.float32)
            return 0
        return lax.fori_loop(0, ncols // 16, col, 0)
    lax.fori_loop(0, nrows, row, 0)


def _fill_ones(ref, nrows, ncols):
    def row(i, _):
        def col(j, _):
            ref[i, pl.ds(j * 16, 16)] = jnp.ones((16,), jnp.float32)
            return 0
        return lax.fori_loop(0, ncols // 16, col, 0)
    lax.fori_loop(0, nrows, row, 0)


def _sc_mesh():
    return plsc.VectorSubcoreMesh(core_axis_name="c", subcore_axis_name="s")


def _deg_body(dst_hbm, dstx_hbm, out_hbm, didx, didx_x, hist_v):
    c = lax.axis_index("c")
    s = lax.axis_index("s")
    wid = s * NC + c

    def zb(k, _):
        hist_v[k, pl.ds(0, 16)] = jnp.zeros((16,), jnp.float32)
        return 0
    lax.fori_loop(0, N // 16, zb, 0)

    ones16 = jnp.ones((16,), jnp.float32)

    def scat(iv):
        rows = lax.shift_right_logical(iv, 4)
        cols = lax.bitwise_and(iv, 15)
        plsc.addupdate_scatter(hist_v, [rows, cols], ones16)

    pltpu.sync_copy(dst_hbm.at[wid], didx)

    def body(j, _):
        def inner(k, _):
            scat(didx[j, pl.ds(k * 16, 16)])
            return 0
        return lax.fori_loop(0, CH // 16, inner, 0)
    lax.fori_loop(0, RPW, body, 0)

    @pl.when(wid < EXTRA)
    def _():
        pltpu.sync_copy(dstx_hbm.at[wid], didx_x)
        def inner(k, _):
            scat(didx_x[0, pl.ds(k * 16, 16)])
            return 0
        lax.fori_loop(0, CH // 16, inner, 0)

    pltpu.sync_copy(hist_v, out_hbm.at[wid])


def _sc_degree(dst3, dstx):
    return pl.kernel(
        _deg_body,
        out_type=jax.ShapeDtypeStruct((NW, N // 16, 16), jnp.float32),
        mesh=_sc_mesh(),
        scratch_types=[
            pltpu.VMEM((RPW, CH), jnp.int32),
            pltpu.VMEM((1, CH), jnp.int32),
            pltpu.VMEM((N // 16, 16), jnp.float32),
        ],
        compiler_params=pltpu.CompilerParams(needs_layout_passes=False),
    )(dst3, dstx)


def _agg_body(g_hbm, src_hbm, dst_hbm, srcx_hbm, dstx_hbm, out_hbm,
              sidx, didx0, didx1, sidx_x, didx_x, rows0, rows1, acc_sh,
              sem0, sem1, *, d):
    c = lax.axis_index("c")
    s = lax.axis_index("s")
    wid = s * NC + c

    _zero_rows(rows0, CH, d)
    for k in range(ROWS_PER_TILE // CH):
        pltpu.sync_copy(rows0, acc_sh.at[pl.ds(s * ROWS_PER_TILE + k * CH, CH)])
    plsc.subcore_barrier()

    pltpu.sync_copy(src_hbm.at[wid], sidx)

    def start(j, rows, didx, sem):
        pltpu.async_copy(dst_hbm.at[wid, pl.ds(j, 1)], didx, sem)
        pltpu.async_copy(g_hbm.at[sidx.at[j]], rows, sem)

    def drain(j, rows, didx, sem):
        pltpu.make_async_copy(dst_hbm.at[wid, pl.ds(j, 1)], didx, sem).wait()
        pltpu.make_async_copy(g_hbm.at[sidx.at[j]], rows, sem).wait()
        pltpu.sync_copy(rows, acc_sh.at[didx.at[0]], add=True)

    start(0, rows0, didx0, sem0)

    def body(j2, _):
        j = 2 * j2
        start(j + 1, rows1, didx1, sem1)
        drain(j, rows0, didx0, sem0)

        @pl.when(j2 < RPW // 2 - 1)
        def _():
            start(j + 2, rows0, didx0, sem0)
        drain(j + 1, rows1, didx1, sem1)
        return 0
    lax.fori_loop(0, RPW // 2, body, 0)

    @pl.when(wid < EXTRA)
    def _():
        pltpu.sync_copy(srcx_hbm.at[wid], sidx_x)
        pltpu.sync_copy(dstx_hbm.at[wid], didx_x)
        pltpu.async_copy(g_hbm.at[sidx_x.at[0]], rows0, sem0).wait()
        pltpu.sync_copy(rows0, acc_sh.at[didx_x.at[0]], add=True)

    plsc.subcore_barrier()

    for k in range(ROWS_PER_TILE // CH):
        base = s * ROWS_PER_TILE + k * CH
        pltpu.sync_copy(acc_sh.at[pl.ds(base, CH)], rows0)
        pltpu.sync_copy(rows0, out_hbm.at[c, pl.ds(base, CH)])


def _sc_agg(g, src3, dst3, srcx, dstx, d):
    cp = None if d == 128 else pltpu.CompilerParams(use_tc_tiling_on_sc=False)
    return pl.kernel(
        functools.partial(_agg_body, d=d),
        compiler_params=cp,
        out_type=jax.ShapeDtypeStruct((NC, NP, d), jnp.float32),
        mesh=_sc_mesh(),
        scratch_types=[
            pltpu.VMEM((RPW, CH), jnp.int32),
            pltpu.VMEM((1, CH), jnp.int32),
            pltpu.VMEM((1, CH), jnp.int32),
            pltpu.VMEM((1, CH), jnp.int32),
            pltpu.VMEM((1, CH), jnp.int32),
            pltpu.VMEM((CH, d), jnp.float32),
            pltpu.VMEM((CH, d), jnp.float32),
            pltpu.VMEM_SHARED((NP, d), jnp.float32),
            pltpu.SemaphoreType.DMA,
            pltpu.SemaphoreType.DMA,
        ],
    )(g, src3, dst3, srcx, dstx)


R = 1000


def _dinv_body(dp_ref, dinv_ref):
    deg = jnp.sum(dp_ref[...], axis=0) + 1.0
    dinv_ref[...] = lax.rsqrt(deg)[:, None]


def _dinv(dp):
    return pl.pallas_call(
        _dinv_body,
        out_shape=jax.ShapeDtypeStruct((N, 1), jnp.float32),
    )(dp)


def _tc1_body(dinv_ref, x_ref, w1_ref, g1_ref):
    g1_ref[...] = jnp.dot(x_ref[...], w1_ref[...],
                          preferred_element_type=jnp.float32) * dinv_ref[...]


def _tc1(dinv, x, W1):
    grid = N // R
    return pl.pallas_call(
        _tc1_body,
        grid=(grid,),
        in_specs=[
            pl.BlockSpec((R, 1), lambda i: (i, 0)),
            pl.BlockSpec((R, 128), lambda i: (i, 0)),
            pl.BlockSpec((128, 128), lambda i: (0, 0)),
        ],
        out_specs=pl.BlockSpec((R, 128), lambda i: (i, 0)),
        out_shape=jax.ShapeDtypeStruct((N, 128), jnp.float32),
    )(dinv, x, W1)


def _tc2_body(dinv_ref, agg_ref, g1_ref, b1_ref, w2_ref, g2_ref):
    dinv = dinv_ref[...]
    out1 = dinv * (agg_ref[0] + agg_ref[1] + g1_ref[...]) + b1_ref[...]
    out1 = jnp.maximum(out1, 0.0)
    g2_ref[...] = jnp.dot(out1, w2_ref[...],
                          preferred_element_type=jnp.float32) * dinv


def _tc2(dinv, agg1, g1, b1, W2):
    grid = N // R
    return pl.pallas_call(
        _tc2_body,
        grid=(grid,),
        in_specs=[
            pl.BlockSpec((R, 1), lambda i: (i, 0)),
            pl.BlockSpec((NC, R, 128), lambda i: (0, i, 0)),
            pl.BlockSpec((R, 128), lambda i: (i, 0)),
            pl.BlockSpec((1, 128), lambda i: (0, 0)),
            pl.BlockSpec((128, 64), lambda i: (0, 0)),
        ],
        out_specs=pl.BlockSpec((R, 64), lambda i: (i, 0)),
        out_shape=jax.ShapeDtypeStruct((N, 64), jnp.float32),
    )(dinv, agg1, g1, b1.reshape(1, 128), W2)


def _tc3_body(dinv_ref, agg_ref, g2_ref, b2_ref, wd_ref, bd_ref, pred_ref):
    dinv = dinv_ref[...]
    z = dinv * (agg_ref[0] + agg_ref[1] + g2_ref[...]) + b2_ref[...]
    m = jnp.max(z, axis=1, keepdims=True)
    lse = jnp.log(jnp.sum(jnp.exp(z - m), axis=1, keepdims=True)) + m
    embeds = z - lse
    logit = jnp.sum(embeds * wd_ref[...], axis=1, keepdims=True) + bd_ref[0, 0]
    pred_ref[...] = jax.nn.sigmoid(logit)


def _tc3(dinv, agg2, g2, b2, Wd, bd):
    grid = N // R
    return pl.pallas_call(
        _tc3_body,
        grid=(grid,),
        in_specs=[
            pl.BlockSpec((R, 1), lambda i: (i, 0)),
            pl.BlockSpec((NC, R, 64), lambda i: (0, i, 0)),
            pl.BlockSpec((R, 64), lambda i: (i, 0)),
            pl.BlockSpec((1, 64), lambda i: (0, 0)),
            pl.BlockSpec((1, 64), lambda i: (0, 0)),
            pl.BlockSpec((1, 1), lambda i: (0, 0)),
        ],
        out_specs=pl.BlockSpec((R, 1), lambda i: (i, 0)),
        out_shape=jax.ShapeDtypeStruct((N, 1), jnp.float32),
    )(dinv, agg2, g2, b2.reshape(1, 64), Wd.reshape(1, 64), bd.reshape(1, 1))


def kernel(x, edge_index, W1, b1, W2, b2, Wd, bd):
    nmain = NW * RPW * CH
    src3 = edge_index[0][:nmain].reshape(NW, RPW, CH)
    dst3 = edge_index[1][:nmain].reshape(NW, RPW, CH)
    srcx = edge_index[0][nmain:].reshape(EXTRA, 1, CH)
    dstx = edge_index[1][nmain:].reshape(EXTRA, 1, CH)
    dp = _sc_degree(dst3, dstx).reshape(NW, N)
    dinv = _dinv(dp)
    g1 = _tc1(dinv, x, W1)
    agg1 = _sc_agg(g1, src3, dst3, srcx, dstx, 128)
    g2 = _tc2(dinv, agg1, g1, b1, W2)
    agg2 = _sc_agg(g2, src3, dst3, srcx, dstx, 64)
    return _tc3(dinv, agg2, g2, b2, Wd, bd)


# speedup vs baseline: 34.6169x; 1.0396x over previous
import functools

import jax
import jax.numpy as jnp
from jax import lax
from jax.experimental import pallas as pl
from jax.experimental.pallas import tpu as pltpu
from jax.experimental.pallas import tpu_sc as plsc

N = 10000
E = 320000
NP = 10240
NC = 2
NS = 16
NW = NC * NS
EPW = E // NW
CH = 128
NROW = E // CH
RPW = NROW // NW
EXTRA = NROW - NW * RPW
ROWS_PER_TILE = NP // NS
DEGW = 16


def _zero_rows(ref, nrows, ncols):
    def row(i, _):
        def col(j, _):
            ref[i, pl.ds(j * 16, 16)] = jnp.zeros((16,), jnp.float32)
            return 0
        return lax.fori_loop(0, ncols // 16, col, 0)
    lax.fori_loop(0, nrows, row, 0)


def _fill_ones(ref, nrows, ncols):
    def row(i, _):
        def col(j, _):
            ref[i, pl.ds(j * 16, 16)] = jnp.ones((16,), jnp.float32)
            return 0
        return lax.fori_loop(0, ncols // 16, col, 0)
    lax.fori_loop(0, nrows, row, 0)


def _sc_mesh():
    return plsc.VectorSubcoreMesh(core_axis_name="c", subcore_axis_name="s")


def _deg_body(dst_hbm, dstx_hbm, out_hbm, didx, didx_x, hist_v):
    c = lax.axis_index("c")
    s = lax.axis_index("s")
    wid = s * NC + c

    def zb(k, _):
        hist_v[k, pl.ds(0, 16)] = jnp.zeros((16,), jnp.float32)
        return 0
    lax.fori_loop(0, N // 16, zb, 0)

    ones16 = jnp.ones((16,), jnp.float32)

    def scat(iv):
        rows = lax.shift_right_logical(iv, 4)
        cols = lax.bitwise_and(iv, 15)
        plsc.addupdate_scatter(hist_v, [rows, cols], ones16)

    pltpu.sync_copy(dst_hbm.at[wid], didx)

    def body(j, _):
        def inner(k, _):
            scat(didx[j, pl.ds(k * 16, 16)])
            return 0
        return lax.fori_loop(0, CH // 16, inner, 0)
    lax.fori_loop(0, RPW, body, 0)

    @pl.when(wid < EXTRA)
    def _():
        pltpu.sync_copy(dstx_hbm.at[wid], didx_x)
        def inner(k, _):
            scat(didx_x[0, pl.ds(k * 16, 16)])
            return 0
        lax.fori_loop(0, CH // 16, inner, 0)

    pltpu.sync_copy(hist_v, out_hbm.at[wid])


def _sc_degree(dst3, dstx):
    return pl.kernel(
        _deg_body,
        out_type=jax.ShapeDtypeStruct((NW, N // 16, 16), jnp.float32),
        mesh=_sc_mesh(),
        scratch_types=[
            pltpu.VMEM((RPW, CH), jnp.int32),
            pltpu.VMEM((1, CH), jnp.int32),
            pltpu.VMEM((N // 16, 16), jnp.float32),
        ],
        compiler_params=pltpu.CompilerParams(needs_layout_passes=False),
    )(dst3, dstx)


def _agg_body(g_hbm, src_hbm, dst_hbm, srcx_hbm, dstx_hbm, out_hbm,
              sidx, didx0, didx1, sidx_x, didx_x, rows0, rows1, acc_sh,
              sem0, sem1, *, d):
    c = lax.axis_index("c")
    s = lax.axis_index("s")
    wid = s * NC + c

    _zero_rows(rows0, CH, d)
    for k in range(ROWS_PER_TILE // CH):
        pltpu.sync_copy(rows0, acc_sh.at[pl.ds(s * ROWS_PER_TILE + k * CH, CH)])
    plsc.subcore_barrier()

    pltpu.sync_copy(src_hbm.at[wid], sidx)

    def start(j, rows, didx, sem):
        pltpu.async_copy(dst_hbm.at[wid, pl.ds(j, 1)], didx, sem)
        pltpu.async_copy(g_hbm.at[sidx.at[j]], rows, sem)

    def drain(j, rows, didx, sem):
        pltpu.make_async_copy(dst_hbm.at[wid, pl.ds(j, 1)], didx, sem).wait()
        pltpu.make_async_copy(g_hbm.at[sidx.at[j]], rows, sem).wait()
        pltpu.sync_copy(rows, acc_sh.at[didx.at[0]], add=True)

    start(0, rows0, didx0, sem0)

    def body(j2, _):
        j = 2 * j2
        start(j + 1, rows1, didx1, sem1)
        drain(j, rows0, didx0, sem0)

        @pl.when(j2 < RPW // 2 - 1)
        def _():
            start(j + 2, rows0, didx0, sem0)
        drain(j + 1, rows1, didx1, sem1)
        return 0
    lax.fori_loop(0, RPW // 2, body, 0)

    @pl.when(wid < EXTRA)
    def _():
        pltpu.sync_copy(srcx_hbm.at[wid], sidx_x)
        pltpu.sync_copy(dstx_hbm.at[wid], didx_x)
        pltpu.async_copy(g_hbm.at[sidx_x.at[0]], rows0, sem0).wait()
        pltpu.sync_copy(rows0, acc_sh.at[didx_x.at[0]], add=True)

    plsc.subcore_barrier()

    for k in range(ROWS_PER_TILE // CH):
        base = s * ROWS_PER_TILE + k * CH
        pltpu.sync_copy(acc_sh.at[pl.ds(base, CH)], rows0)
        pltpu.sync_copy(rows0, out_hbm.at[c, pl.ds(base, CH)])


def _sc_agg(g, src3, dst3, srcx, dstx, d):
    cp = None if d == 128 else pltpu.CompilerParams(use_tc_tiling_on_sc=False)
    return pl.kernel(
        functools.partial(_agg_body, d=d),
        compiler_params=cp,
        out_type=jax.ShapeDtypeStruct((NC, NP, d), jnp.float32),
        mesh=_sc_mesh(),
        scratch_types=[
            pltpu.VMEM((RPW, CH), jnp.int32),
            pltpu.VMEM((1, CH), jnp.int32),
            pltpu.VMEM((1, CH), jnp.int32),
            pltpu.VMEM((1, CH), jnp.int32),
            pltpu.VMEM((1, CH), jnp.int32),
            pltpu.VMEM((CH, d), jnp.float32),
            pltpu.VMEM((CH, d), jnp.float32),
            pltpu.VMEM_SHARED((NP, d), jnp.float32),
            pltpu.SemaphoreType.DMA,
            pltpu.SemaphoreType.DMA,
        ],
    )(g, src3, dst3, srcx, dstx)


R = 1000


def _tc1_body(x_ref, w1_ref, h1_ref):
    h1_ref[...] = jnp.dot(x_ref[...], w1_ref[...],
                          preferred_element_type=jnp.float32)


def _tc1(x, W1):
    grid = N // R
    return pl.pallas_call(
        _tc1_body,
        grid=(grid,),
        in_specs=[
            pl.BlockSpec((R, 128), lambda i: (i, 0)),
            pl.BlockSpec((128, 128), lambda i: (0, 0)),
        ],
        out_specs=pl.BlockSpec((R, 128), lambda i: (i, 0)),
        out_shape=jax.ShapeDtypeStruct((N, 128), jnp.float32),
    )(x, W1)


def _dinv_scale_body(dp_ref, h1_ref, dinv_ref, g1_ref):
    deg = jnp.sum(dp_ref[...], axis=0) + 1.0
    dinv = lax.rsqrt(deg)[:, None]
    dinv_ref[...] = dinv
    g1_ref[...] = h1_ref[...] * dinv


def _dinv_scale(dp, h1):
    return pl.pallas_call(
        _dinv_scale_body,
        out_shape=[
            jax.ShapeDtypeStruct((N, 1), jnp.float32),
            jax.ShapeDtypeStruct((N, 128), jnp.float32),
        ],
    )(dp, h1)


def _tc2_body(dinv_ref, agg_ref, g1_ref, b1_ref, w2_ref, g2_ref):
    dinv = dinv_ref[...]
    out1 = dinv * (agg_ref[0] + agg_ref[1] + g1_ref[...]) + b1_ref[...]
    out1 = jnp.maximum(out1, 0.0)
    g2_ref[...] = jnp.dot(out1, w2_ref[...],
                          preferred_element_type=jnp.float32) * dinv


def _tc2(dinv, agg1, g1, b1, W2):
    grid = N // R
    return pl.pallas_call(
        _tc2_body,
        grid=(grid,),
        in_specs=[
            pl.BlockSpec((R, 1), lambda i: (i, 0)),
            pl.BlockSpec((NC, R, 128), lambda i: (0, i, 0)),
            pl.BlockSpec((R, 128), lambda i: (i, 0)),
            pl.BlockSpec((1, 128), lambda i: (0, 0)),
            pl.BlockSpec((128, 64), lambda i: (0, 0)),
        ],
        out_specs=pl.BlockSpec((R, 64), lambda i: (i, 0)),
        out_shape=jax.ShapeDtypeStruct((N, 64), jnp.float32),
    )(dinv, agg1, g1, b1.reshape(1, 128), W2)


def _tc3_body(dinv_ref, agg_ref, g2_ref, b2_ref, wd_ref, bd_ref, pred_ref):
    dinv = dinv_ref[...]
    z = dinv * (agg_ref[0] + agg_ref[1] + g2_ref[...]) + b2_ref[...]
    m = jnp.max(z, axis=1, keepdims=True)
    lse = jnp.log(jnp.sum(jnp.exp(z - m), axis=1, keepdims=True)) + m
    embeds = z - lse
    logit = jnp.sum(embeds * wd_ref[...], axis=1, keepdims=True) + bd_ref[0, 0]
    pred_ref[...] = jax.nn.sigmoid(logit)


def _tc3(dinv, agg2, g2, b2, Wd, bd):
    grid = N // R
    return pl.pallas_call(
        _tc3_body,
        grid=(grid,),
        in_specs=[
            pl.BlockSpec((R, 1), lambda i: (i, 0)),
            pl.BlockSpec((NC, R, 64), lambda i: (0, i, 0)),
            pl.BlockSpec((R, 64), lambda i: (i, 0)),
            pl.BlockSpec((1, 64), lambda i: (0, 0)),
            pl.BlockSpec((1, 64), lambda i: (0, 0)),
            pl.BlockSpec((1, 1), lambda i: (0, 0)),
        ],
        out_specs=pl.BlockSpec((R, 1), lambda i: (i, 0)),
        out_shape=jax.ShapeDtypeStruct((N, 1), jnp.float32),
    )(dinv, agg2, g2, b2.reshape(1, 64), Wd.reshape(1, 64), bd.reshape(1, 1))


def kernel(x, edge_index, W1, b1, W2, b2, Wd, bd):
    nmain = NW * RPW * CH
    src3 = edge_index[0][:nmain].reshape(NW, RPW, CH)
    dst3 = edge_index[1][:nmain].reshape(NW, RPW, CH)
    srcx = edge_index[0][nmain:].reshape(EXTRA, 1, CH)
    dstx = edge_index[1][nmain:].reshape(EXTRA, 1, CH)
    dp = _sc_degree(dst3, dstx).reshape(NW, N)
    h1 = _tc1(x, W1)
    dinv, g1 = _dinv_scale(dp, h1)
    agg1 = _sc_agg(g1, src3, dst3, srcx, dstx, 128)
    g2 = _tc2(dinv, agg1, g1, b1, W2)
    agg2 = _sc_agg(g2, src3, dst3, srcx, dstx, 64)
    return _tc3(dinv, agg2, g2, b2, Wd, bd)
